# baseline (speedup 1.0000x reference)
import functools

import jax
import jax.numpy as jnp
from jax import lax
from jax.experimental import pallas as pl
from jax.experimental.pallas import tpu as pltpu
from jax.experimental.pallas import tpu_sc as plsc

VOCAB = 1000000
EMBED_DIM = 16
BATCH = 16384
N_FIELDS = 26

_NW = 32
_B_PER_W = BATCH // _NW
_CB = 64
_NCHUNK = _B_PER_W // _CB
_CIDX = _CB * N_FIELDS
_RP = 17


def _sig_kernel(table_hbm, xq_hbm, out_hbm, idx4_v, idx_v, rows_v, pad_v,
                stg_v, sem):
    wid = lax.axis_index("s") * 2 + lax.axis_index("c")
    lanes = jnp.arange(16, dtype=jnp.int32)
    for c in range(_NCHUNK):
        b0 = wid * _B_PER_W + c * _CB
        jt = wid * 4 + c // 2
        c0 = (c % 2) * _CB
        pltpu.sync_copy(xq_hbm.at[:, jt, :, pl.ds(c0, _CB)], idx4_v)
        for f in range(N_FIELDS):
            for k in range(_CB // 16):
                idx_v[pl.ds(f * _CB + k * 16, 16)] = \
                    idx4_v[f // 8, f % 8, pl.ds(k * 16, 16)]
        pltpu.async_copy(table_hbm.at[idx_v], rows_v, sem).wait()

        def sig_body(j, carry):
            r = rows_v[j]
            pad_v[pl.ds(j * _RP, 16)] = 1.0 / (1.0 + jnp.exp(-r))
            return carry

        lax.fori_loop(0, _CIDX, sig_body, 0)

        def tr_body(f, carry):
            for q in range(_CB // 16):
                base = (f * _CB + q * 16 + lanes) * _RP
                for e in range(EMBED_DIM):
                    v = plsc.load_gather(pad_v, [base + e])
                    stg_v[f, e, pl.ds(q * 16, 16)] = v
            return carry

        lax.fori_loop(0, N_FIELDS, tr_body, 0)
        pltpu.sync_copy(stg_v, out_hbm.at[:, :, pl.ds(b0, _CB)])


@jax.jit
def _run(table, xq):
    mesh = plsc.VectorSubcoreMesh(core_axis_name="c", subcore_axis_name="s")
    f = functools.partial(
        pl.kernel,
        mesh=mesh,
        out_type=jax.ShapeDtypeStruct((N_FIELDS, EMBED_DIM, BATCH), jnp.float32),
        scratch_types=[
            pltpu.VMEM((4, 8, _CB), jnp.int32),
            pltpu.VMEM((_CIDX,), jnp.int32),
            pltpu.VMEM((_CIDX, EMBED_DIM), jnp.float32),
            pltpu.VMEM((_CIDX * _RP,), jnp.float32),
            pltpu.VMEM((N_FIELDS, EMBED_DIM, _CB), jnp.float32),
            pltpu.SemaphoreType.DMA,
        ],
        compiler_params=pltpu.CompilerParams(
            use_tc_tiling_on_sc=False, needs_layout_passes=False),
    )(_sig_kernel)
    return f(table, xq)


def kernel(table, x):
    xp = jnp.pad(x, ((0, 0), (0, 32 - N_FIELDS)))
    xq = xp.T.reshape(4, 8, 128, 128).transpose(0, 2, 1, 3)
    out_t = _run(table, xq)
    return jnp.transpose(out_t, (2, 0, 1))
